# baseline (speedup 1.0000x reference)
import functools

import jax
import jax.numpy as jnp
from jax import lax
from jax.experimental import pallas as pl
from jax.experimental.pallas import tpu as pltpu
from jax.experimental.pallas import tpu_sc as plsc

E = 8
TOPK = 2
T = 2048
D = 2048
DFF = 768
TM = 256
NCHUNK = T // TM
NPAIR = 2 * TM
TG = 128
P = T * TOPK + E * TG
NT = P // TG

NC = 2
NS = 16
NW = NC * NS


def _top2(x, gate_w):
    logits = lax.dot_general(x, gate_w, (((1,), (1,)), ((), ())),
                             preferred_element_type=jnp.float32)
    logits = logits.astype(jnp.bfloat16).astype(jnp.float32)
    m = jnp.max(logits, axis=1, keepdims=True)
    ex = jnp.exp(logits - m)
    probs = ex / jnp.sum(ex, axis=1, keepdims=True)
    idx = lax.broadcasted_iota(jnp.int32, probs.shape, 1)
    big = jnp.int32(E)
    m1 = jnp.max(probs, axis=1, keepdims=True)
    i1 = jnp.min(jnp.where(probs == m1, idx, big), axis=1, keepdims=True)
    probs2 = jnp.where(idx == i1, -1.0, probs)
    m2 = jnp.max(probs2, axis=1, keepdims=True)
    i2 = jnp.min(jnp.where(probs2 == m2, idx, big), axis=1, keepdims=True)
    s = m1 + m2
    c1 = (m1 / s).astype(jnp.bfloat16).astype(jnp.float32)
    c2 = (m2 / s).astype(jnp.bfloat16).astype(jnp.float32)
    oh1 = jnp.where(idx == i1, 1.0, 0.0)
    oh2 = jnp.where(idx == i2, 1.0, 0.0)
    return c1, c2, oh1, oh2


def _route_kernel(x_ref, gate_ref, pos_ref, w_ref, te_ref,
                  carry_ref, base_ref):
    p = pl.program_id(0)
    c = pl.program_id(1)
    c1, c2, oh1, oh2 = _top2(x_ref[...], gate_ref[...])
    oh = jnp.concatenate([oh1, oh2], axis=0)

    @pl.when(c == 0)
    def _():
        @pl.when(p == 0)
        def _():
            base_ref[...] = jnp.zeros_like(base_ref)

        @pl.when(p == 1)
        def _():
            totals = carry_ref[...]
            padded = jnp.ceil(totals / TG) * TG
            eidx = lax.broadcasted_iota(jnp.int32, (E, E), 0)
            fidx = lax.broadcasted_iota(jnp.int32, (E, E), 1)
            ut = jnp.where(eidx < fidx, 1.0, 0.0)
            base_ref[...] = lax.dot_general(
                padded, ut, (((1,), (0,)), ((), ())),
                preferred_element_type=jnp.float32)

        carry_ref[...] = jnp.zeros_like(carry_ref)

    r = lax.broadcasted_iota(jnp.int32, (NPAIR, NPAIR), 0)
    cc = lax.broadcasted_iota(jnp.int32, (NPAIR, NPAIR), 1)
    lt = jnp.where(r > cc, 1.0, 0.0)
    excl = lax.dot_general(lt, oh, (((1,), (0,)), ((), ())),
                           preferred_element_type=jnp.float32)
    excl = excl + carry_ref[...]
    carry_ref[...] = carry_ref[...] + jnp.sum(oh, axis=0, keepdims=True)

    pos = jnp.sum(oh * (excl + base_ref[...]), axis=1, keepdims=True)
    pos_i = pos.astype(jnp.int32)
    pos_ref[...] = jnp.broadcast_to(pos_i, (NPAIR, E))
    w_ref[0] = jnp.concatenate([c1, c2], axis=1)

    @pl.when((p == 1) & (c == NCHUNK - 1))
    def _():
        jj = (lax.broadcasted_iota(jnp.int32, (NT, E), 0) * TG
              ).astype(jnp.float32)
        cnt = jnp.sum(jnp.where(base_ref[...] <= jj, 1, 0),
                      axis=1, keepdims=True) - 1
        te_ref[...] = jnp.broadcast_to(cnt, (NT, E))


def _route(x, gate_w):
    return pl.pallas_call(
        _route_kernel,
        grid=(2, NCHUNK),
        in_specs=[
            pl.BlockSpec((TM, D), lambda p, c: (c, 0)),
            pl.BlockSpec((E, D), lambda p, c: (0, 0)),
        ],
        out_specs=[
            pl.BlockSpec((NPAIR, E), lambda p, c: (c, 0)),
            pl.BlockSpec((1, TM, TOPK), lambda p, c: (c, 0, 0)),
            pl.BlockSpec((NT, E), lambda p, c: (0, 0)),
        ],
        out_shape=[
            jax.ShapeDtypeStruct((T * TOPK, E), jnp.int32),
            jax.ShapeDtypeStruct((NCHUNK, TM, TOPK), jnp.float32),
            jax.ShapeDtypeStruct((NT, E), jnp.int32),
        ],
        scratch_shapes=[
            pltpu.VMEM((1, E), jnp.float32),
            pltpu.VMEM((1, E), jnp.float32),
        ],
        compiler_params=pltpu.CompilerParams(
            dimension_semantics=("arbitrary", "arbitrary")),
    )(x, gate_w)


PAIRS_PER_W = (T * TOPK) // NW
SCHUNK = 64


def _scatter_x_kernel(pos_hbm, x_hbm, xs_hbm, idx_v, rows_v, sem):
    wid = lax.axis_index("s") * NC + lax.axis_index("c")
    for j in range(PAIRS_PER_W // SCHUNK):
        f0 = wid * PAIRS_PER_W + j * SCHUNK
        tok0 = ((f0 >> 9) << 8) + (f0 & 255)
        pltpu.sync_copy(pos_hbm.at[pl.ds(f0, SCHUNK)], idx_v)
        pltpu.sync_copy(x_hbm.at[pl.ds(tok0, SCHUNK)], rows_v)
        pltpu.async_copy(rows_v, xs_hbm.at[idx_v], sem).wait()


def _scatter_x(pos_flat, x3i):
    return pl.kernel(
        _scatter_x_kernel,
        out_type=jax.ShapeDtypeStruct((P, 8, 128), jnp.int32),
        mesh=plsc.VectorSubcoreMesh(core_axis_name="c", subcore_axis_name="s"),
        scratch_types=[
            pltpu.VMEM((SCHUNK,), jnp.int32),
            pltpu.VMEM((SCHUNK, 8, 128), jnp.int32),
            pltpu.SemaphoreType.DMA,
        ],
    )(pos_flat, x3i)


def _gemm_kernel(te_ref, xs_ref, wg_ref, wu_ref, wd_ref, ys_ref):
    xt = xs_ref[...]
    g = lax.dot_general(xt, wg_ref[0], (((1,), (1,)), ((), ())),
                        preferred_element_type=jnp.float32)
    u = lax.dot_general(xt, wu_ref[0], (((1,), (1,)), ((), ())),
                        preferred_element_type=jnp.float32)
    g16 = g.astype(jnp.bfloat16)
    u16 = u.astype(jnp.bfloat16)
    sig = 1.0 / (1.0 + jnp.exp(-g16.astype(jnp.float32)))
    h = ((g16.astype(jnp.float32) * sig).astype(jnp.bfloat16) * u16)
    y = lax.dot_general(h, wd_ref[0], (((1,), (1,)), ((), ())),
                        preferred_element_type=jnp.float32)
    ys_ref[...] = y.astype(jnp.bfloat16)


def _grouped_gemm(te, xs, w_gate, w_up, w_down):
    grid_spec = pltpu.PrefetchScalarGridSpec(
        num_scalar_prefetch=1,
        grid=(NT,),
        in_specs=[
            pl.BlockSpec((TG, D), lambda i, te: (i, 0)),
            pl.BlockSpec((1, DFF, D), lambda i, te: (te[i], 0, 0)),
            pl.BlockSpec((1, DFF, D), lambda i, te: (te[i], 0, 0)),
            pl.BlockSpec((1, D, DFF), lambda i, te: (te[i], 0, 0)),
        ],
        out_specs=pl.BlockSpec((TG, D), lambda i, te: (i, 0)),
    )
    return pl.pallas_call(
        _gemm_kernel,
        grid_spec=grid_spec,
        out_shape=jax.ShapeDtypeStruct((P, D), jnp.bfloat16),
        compiler_params=pltpu.CompilerParams(
            dimension_semantics=("arbitrary",)),
    )(te, xs, w_gate, w_up, w_down)


TOK_PER_W = T // NW


def _gather_y_kernel(idx0_hbm, idx1_hbm, ys_hbm, r0_hbm, r1_hbm,
                     idx_v, rows_v, sem):
    wid = lax.axis_index("s") * NC + lax.axis_index("c")
    base = wid * TOK_PER_W
    for src, dst in ((idx0_hbm, r0_hbm), (idx1_hbm, r1_hbm)):
        pltpu.sync_copy(src.at[pl.ds(base, TOK_PER_W)], idx_v)
        pltpu.async_copy(ys_hbm.at[idx_v], rows_v, sem).wait()
        pltpu.sync_copy(rows_v, dst.at[pl.ds(base, TOK_PER_W)])


def _gather_y(idx0, idx1, ys3i):
    return pl.kernel(
        _gather_y_kernel,
        out_type=[
            jax.ShapeDtypeStruct((T, 8, 128), jnp.int32),
            jax.ShapeDtypeStruct((T, 8, 128), jnp.int32),
        ],
        mesh=plsc.VectorSubcoreMesh(core_axis_name="c", subcore_axis_name="s"),
        scratch_types=[
            pltpu.VMEM((TOK_PER_W,), jnp.int32),
            pltpu.VMEM((TOK_PER_W, 8, 128), jnp.int32),
            pltpu.SemaphoreType.DMA,
        ],
    )(idx0, idx1, ys3i)


def _combine_kernel(r0_ref, r1_ref, w_ref, out_ref):
    w0 = w_ref[:, 0:1].astype(jnp.bfloat16)
    w1 = w_ref[:, 1:2].astype(jnp.bfloat16)
    out_ref[...] = w0 * r0_ref[...] + w1 * r1_ref[...]


def _combine(r0, r1, wt):
    return pl.pallas_call(
        _combine_kernel,
        grid=(NCHUNK,),
        in_specs=[
            pl.BlockSpec((TM, D), lambda c: (c, 0)),
            pl.BlockSpec((TM, D), lambda c: (c, 0)),
            pl.BlockSpec((TM, TOPK), lambda c: (c, 0)),
        ],
        out_specs=pl.BlockSpec((TM, D), lambda c: (c, 0)),
        out_shape=jax.ShapeDtypeStruct((T, D), jnp.bfloat16),
        compiler_params=pltpu.CompilerParams(
            dimension_semantics=("arbitrary",)),
    )(r0, r1, wt)


def kernel(hidden_states, gate_w, w_gate, w_up, w_down):
    b, s, d = hidden_states.shape
    x = hidden_states.reshape(-1, d)

    pos_rep, w_t, te_rep = _route(x, gate_w)
    pos_flat = pos_rep[:, 0]
    pos3 = pos_flat.reshape(NCHUNK, TOPK, TM)
    idx0 = pos3[:, 0, :].reshape(T)
    idx1 = pos3[:, 1, :].reshape(T)
    wt = w_t.reshape(T, TOPK)
    te = te_rep[:, 0]

    x3i = lax.bitcast_convert_type(
        x.reshape(T, 8, 128, 2), jnp.int32)
    xs3i = _scatter_x(pos_flat, x3i)
    xs = lax.bitcast_convert_type(
        xs3i, jnp.bfloat16).reshape(P, D)
    ys = _grouped_gemm(te, xs, w_gate, w_up, w_down)
    ys3i = lax.bitcast_convert_type(
        ys.reshape(P, 8, 128, 2), jnp.int32)
    r0i, r1i = _gather_y(idx0, idx1, ys3i)
    r0 = lax.bitcast_convert_type(r0i, jnp.bfloat16).reshape(T, D)
    r1 = lax.bitcast_convert_type(r1i, jnp.bfloat16).reshape(T, D)
    out = _combine(r0, r1, wt)
    return out.reshape(b, s, d)


# speedup vs baseline: 4.0077x; 4.0077x over previous
import functools

import jax
import jax.numpy as jnp
from jax import lax
from jax.experimental import pallas as pl
from jax.experimental.pallas import tpu as pltpu
from jax.experimental.pallas import tpu_sc as plsc

E = 8
TOPK = 2
T = 2048
D = 2048
DFF = 768
TM = 256
NCHUNK = T // TM
NPAIR = 2 * TM
TG = 128
P = T * TOPK + E * TG
NT = P // TG

NC = 2
NS = 16
NW = NC * NS


def _top2(x, gate_w):
    logits = lax.dot_general(x, gate_w, (((1,), (1,)), ((), ())),
                             preferred_element_type=jnp.float32)
    logits = logits.astype(jnp.bfloat16).astype(jnp.float32)
    m = jnp.max(logits, axis=1, keepdims=True)
    ex = jnp.exp(logits - m)
    probs = ex / jnp.sum(ex, axis=1, keepdims=True)
    idx = lax.broadcasted_iota(jnp.int32, probs.shape, 1)
    big = jnp.int32(E)
    m1 = jnp.max(probs, axis=1, keepdims=True)
    i1 = jnp.min(jnp.where(probs == m1, idx, big), axis=1, keepdims=True)
    probs2 = jnp.where(idx == i1, -1.0, probs)
    m2 = jnp.max(probs2, axis=1, keepdims=True)
    i2 = jnp.min(jnp.where(probs2 == m2, idx, big), axis=1, keepdims=True)
    s = m1 + m2
    c1 = (m1 / s).astype(jnp.bfloat16).astype(jnp.float32)
    c2 = (m2 / s).astype(jnp.bfloat16).astype(jnp.float32)
    oh1 = jnp.where(idx == i1, 1.0, 0.0)
    oh2 = jnp.where(idx == i2, 1.0, 0.0)
    return c1, c2, oh1, oh2


def _route_kernel(x_ref, gate_ref, pos_ref, w_ref, te_ref, live_ref,
                  carry_ref, base_ref):
    p = pl.program_id(0)
    c = pl.program_id(1)
    c1, c2, oh1, oh2 = _top2(x_ref[...], gate_ref[...])
    oh = jnp.concatenate([oh1, oh2], axis=0)

    @pl.when(c == 0)
    def _():
        @pl.when(p == 0)
        def _():
            base_ref[...] = jnp.zeros_like(base_ref)

        @pl.when(p == 1)
        def _():
            totals = carry_ref[...]
            padded = jnp.ceil(totals / TG) * TG
            eidx = lax.broadcasted_iota(jnp.int32, (E, E), 0)
            fidx = lax.broadcasted_iota(jnp.int32, (E, E), 1)
            ut = jnp.where(eidx < fidx, 1.0, 0.0)
            base_ref[...] = lax.dot_general(
                padded, ut, (((1,), (0,)), ((), ())),
                preferred_element_type=jnp.float32)

        carry_ref[...] = jnp.zeros_like(carry_ref)

    r = lax.broadcasted_iota(jnp.int32, (NPAIR, NPAIR), 0)
    cc = lax.broadcasted_iota(jnp.int32, (NPAIR, NPAIR), 1)
    lt = jnp.where(r > cc, 1.0, 0.0)
    excl = lax.dot_general(lt, oh, (((1,), (0,)), ((), ())),
                           preferred_element_type=jnp.float32)
    excl = excl + carry_ref[...]
    carry_ref[...] = carry_ref[...] + jnp.sum(oh, axis=0, keepdims=True)

    pos = jnp.sum(oh * (excl + base_ref[...]), axis=1, keepdims=True)
    pos_i = pos.astype(jnp.int32)
    pos_ref[...] = jnp.broadcast_to(pos_i, (NPAIR, E))
    w_ref[0] = jnp.concatenate([c1, c2], axis=1)

    @pl.when((p == 1) & (c == NCHUNK - 1))
    def _():
        jj = (lax.broadcasted_iota(jnp.int32, (NT, E), 0) * TG
              ).astype(jnp.float32)
        cnt = jnp.sum(jnp.where(base_ref[...] <= jj, 1, 0),
                      axis=1, keepdims=True) - 1
        te_ref[...] = jnp.broadcast_to(cnt, (NT, E))
        tot = carry_ref[...]
        hit = jnp.where((jj >= base_ref[...]) & (jj < base_ref[...] + tot),
                        1, 0)
        live = jnp.sum(hit, axis=1, keepdims=True)
        live_ref[...] = jnp.broadcast_to(live, (NT, E))


def _route(x, gate_w):
    return pl.pallas_call(
        _route_kernel,
        grid=(2, NCHUNK),
        in_specs=[
            pl.BlockSpec((TM, D), lambda p, c: (c, 0)),
            pl.BlockSpec((E, D), lambda p, c: (0, 0)),
        ],
        out_specs=[
            pl.BlockSpec((NPAIR, E), lambda p, c: (c, 0)),
            pl.BlockSpec((1, TM, TOPK), lambda p, c: (c, 0, 0)),
            pl.BlockSpec((NT, E), lambda p, c: (0, 0)),
            pl.BlockSpec((NT, E), lambda p, c: (0, 0)),
        ],
        out_shape=[
            jax.ShapeDtypeStruct((T * TOPK, E), jnp.int32),
            jax.ShapeDtypeStruct((NCHUNK, TM, TOPK), jnp.float32),
            jax.ShapeDtypeStruct((NT, E), jnp.int32),
            jax.ShapeDtypeStruct((NT, E), jnp.int32),
        ],
        scratch_shapes=[
            pltpu.VMEM((1, E), jnp.float32),
            pltpu.VMEM((1, E), jnp.float32),
        ],
        compiler_params=pltpu.CompilerParams(
            dimension_semantics=("arbitrary", "arbitrary")),
    )(x, gate_w)


PAIRS_PER_W = (T * TOPK) // NW
SCHUNK = 64


def _scatter_x_kernel(pos_hbm, x_hbm, xs_hbm, idx_v, rows_v, sem):
    x_i = x_hbm.bitcast(jnp.int32)
    xs_i = xs_hbm.bitcast(jnp.int32)
    wid = lax.axis_index("s") * NC + lax.axis_index("c")
    for j in range(PAIRS_PER_W // SCHUNK):
        f0 = wid * PAIRS_PER_W + j * SCHUNK
        tok0 = ((f0 >> 9) << 8) + (f0 & 255)
        pltpu.sync_copy(pos_hbm.at[pl.ds(f0, SCHUNK)], idx_v)
        pltpu.sync_copy(x_i.at[pl.ds(tok0, SCHUNK)], rows_v)
        pltpu.async_copy(rows_v, xs_i.at[idx_v], sem).wait()


def _scatter_x(pos_flat, x3):
    return pl.kernel(
        _scatter_x_kernel,
        out_type=jax.ShapeDtypeStruct((P, 16, 128), jnp.bfloat16),
        mesh=plsc.VectorSubcoreMesh(core_axis_name="c", subcore_axis_name="s"),
        scratch_types=[
            pltpu.VMEM((SCHUNK,), jnp.int32),
            pltpu.VMEM((SCHUNK, 8, 128), jnp.int32),
            pltpu.SemaphoreType.DMA,
        ],
    )(pos_flat, x3)


def _gemm_kernel(te_ref, live_ref, xs_ref, wg_ref, wu_ref, wd_ref, ys_ref):
    i = pl.program_id(0)

    @pl.when(live_ref[i] != 0)
    def _():
        xt = xs_ref[...].reshape(TG, D)
        g = lax.dot_general(xt, wg_ref[0], (((1,), (1,)), ((), ())),
                            preferred_element_type=jnp.float32)
        u = lax.dot_general(xt, wu_ref[0], (((1,), (1,)), ((), ())),
                            preferred_element_type=jnp.float32)
        g16 = g.astype(jnp.bfloat16)
        u16 = u.astype(jnp.bfloat16)
        sig = 1.0 / (1.0 + jnp.exp(-g16.astype(jnp.float32)))
        h = ((g16.astype(jnp.float32) * sig).astype(jnp.bfloat16) * u16)
        y = lax.dot_general(h, wd_ref[0], (((1,), (1,)), ((), ())),
                            preferred_element_type=jnp.float32)
        ys_ref[...] = y.astype(jnp.bfloat16).reshape(TG, 16, 128)


def _grouped_gemm(te, live, xs3, w_gate, w_up, w_down):
    grid_spec = pltpu.PrefetchScalarGridSpec(
        num_scalar_prefetch=2,
        grid=(NT,),
        in_specs=[
            pl.BlockSpec((TG, 16, 128), lambda i, te, lv: (i, 0, 0)),
            pl.BlockSpec((1, DFF, D), lambda i, te, lv: (te[i], 0, 0)),
            pl.BlockSpec((1, DFF, D), lambda i, te, lv: (te[i], 0, 0)),
            pl.BlockSpec((1, D, DFF), lambda i, te, lv: (te[i], 0, 0)),
        ],
        out_specs=pl.BlockSpec((TG, 16, 128), lambda i, te, lv: (i, 0, 0)),
    )
    return pl.pallas_call(
        _gemm_kernel,
        grid_spec=grid_spec,
        out_shape=jax.ShapeDtypeStruct((P, 16, 128), jnp.bfloat16),
        compiler_params=pltpu.CompilerParams(
            dimension_semantics=("arbitrary",)),
    )(te, live, xs3, w_gate, w_up, w_down)


TOK_PER_W = T // NW


def _gather_y_kernel(idx0_hbm, idx1_hbm, ys_hbm, r0_hbm, r1_hbm,
                     idx_v, rows_v, sem):
    ys_i = ys_hbm.bitcast(jnp.int32)
    wid = lax.axis_index("s") * NC + lax.axis_index("c")
    base = wid * TOK_PER_W
    for src, dst in ((idx0_hbm, r0_hbm), (idx1_hbm, r1_hbm)):
        pltpu.sync_copy(src.at[pl.ds(base, TOK_PER_W)], idx_v)
        pltpu.async_copy(ys_i.at[idx_v], rows_v, sem).wait()
        pltpu.sync_copy(rows_v, dst.bitcast(jnp.int32).at[pl.ds(base, TOK_PER_W)])


def _gather_y(idx0, idx1, ys3):
    return pl.kernel(
        _gather_y_kernel,
        out_type=[
            jax.ShapeDtypeStruct((T, 16, 128), jnp.bfloat16),
            jax.ShapeDtypeStruct((T, 16, 128), jnp.bfloat16),
        ],
        mesh=plsc.VectorSubcoreMesh(core_axis_name="c", subcore_axis_name="s"),
        scratch_types=[
            pltpu.VMEM((TOK_PER_W,), jnp.int32),
            pltpu.VMEM((TOK_PER_W, 8, 128), jnp.int32),
            pltpu.SemaphoreType.DMA,
        ],
    )(idx0, idx1, ys3)


def _combine_kernel(r0_ref, r1_ref, w_ref, out_ref):
    w0 = w_ref[:, 0:1].astype(jnp.bfloat16)[:, :, None]
    w1 = w_ref[:, 1:2].astype(jnp.bfloat16)[:, :, None]
    out = w0 * r0_ref[...] + w1 * r1_ref[...]
    out_ref[...] = out.reshape(TM, D)


def _combine(r0, r1, wt):
    return pl.pallas_call(
        _combine_kernel,
        grid=(NCHUNK,),
        in_specs=[
            pl.BlockSpec((TM, 16, 128), lambda c: (c, 0, 0)),
            pl.BlockSpec((TM, 16, 128), lambda c: (c, 0, 0)),
            pl.BlockSpec((TM, TOPK), lambda c: (c, 0)),
        ],
        out_specs=pl.BlockSpec((TM, D), lambda c: (c, 0)),
        out_shape=jax.ShapeDtypeStruct((T, D), jnp.bfloat16),
        compiler_params=pltpu.CompilerParams(
            dimension_semantics=("arbitrary",)),
    )(r0, r1, wt)


def kernel(hidden_states, gate_w, w_gate, w_up, w_down):
    b, s, d = hidden_states.shape
    x = hidden_states.reshape(-1, d)

    pos_rep, w_t, te_rep, live_rep = _route(x, gate_w)
    pos_flat = pos_rep[:, 0]
    pos3 = pos_flat.reshape(NCHUNK, TOPK, TM)
    idx0 = pos3[:, 0, :].reshape(T)
    idx1 = pos3[:, 1, :].reshape(T)
    wt = w_t.reshape(T, TOPK)
    te = te_rep[:, 0]
    live = live_rep[:, 0]

    x3 = x.reshape(T, 16, 128)
    xs3 = _scatter_x(pos_flat, x3)
    ys3 = _grouped_gemm(te, live, xs3, w_gate, w_up, w_down)
    r0, r1 = _gather_y(idx0, idx1, ys3)
    out = _combine(r0, r1, wt)
    return out.reshape(b, s, d)


# speedup vs baseline: 5.3807x; 1.3426x over previous
import functools

import jax
import jax.numpy as jnp
from jax import lax
from jax.experimental import pallas as pl
from jax.experimental.pallas import tpu as pltpu
from jax.experimental.pallas import tpu_sc as plsc

E = 8
TOPK = 2
T = 2048
D = 2048
DFF = 768
TM = 256
NCHUNK = T // TM
NPAIR = 2 * TM
TG = 256
P = T * TOPK + E * TG
NT = P // TG

NC = 2
NS = 16
NW = NC * NS


def _top2(x, gate_w):
    logits = lax.dot_general(x, gate_w, (((1,), (1,)), ((), ())),
                             preferred_element_type=jnp.float32)
    logits = logits.astype(jnp.bfloat16).astype(jnp.float32)
    m = jnp.max(logits, axis=1, keepdims=True)
    ex = jnp.exp(logits - m)
    probs = ex / jnp.sum(ex, axis=1, keepdims=True)
    idx = lax.broadcasted_iota(jnp.int32, probs.shape, 1)
    big = jnp.int32(E)
    m1 = jnp.max(probs, axis=1, keepdims=True)
    i1 = jnp.min(jnp.where(probs == m1, idx, big), axis=1, keepdims=True)
    probs2 = jnp.where(idx == i1, -1.0, probs)
    m2 = jnp.max(probs2, axis=1, keepdims=True)
    i2 = jnp.min(jnp.where(probs2 == m2, idx, big), axis=1, keepdims=True)
    s = m1 + m2
    c1 = (m1 / s).astype(jnp.bfloat16).astype(jnp.float32)
    c2 = (m2 / s).astype(jnp.bfloat16).astype(jnp.float32)
    oh1 = jnp.where(idx == i1, 1.0, 0.0)
    oh2 = jnp.where(idx == i2, 1.0, 0.0)
    return c1, c2, oh1, oh2


def _route_kernel(x_ref, gate_ref, pos_ref, w_ref, te_ref, live_ref,
                  carry_ref, base_ref):
    p = pl.program_id(0)
    c = pl.program_id(1)
    c1, c2, oh1, oh2 = _top2(x_ref[...], gate_ref[...])
    oh = jnp.concatenate([oh1, oh2], axis=0)

    @pl.when(c == 0)
    def _():
        @pl.when(p == 0)
        def _():
            base_ref[...] = jnp.zeros_like(base_ref)

        @pl.when(p == 1)
        def _():
            totals = carry_ref[...]
            padded = jnp.ceil(totals / TG) * TG
            eidx = lax.broadcasted_iota(jnp.int32, (E, E), 0)
            fidx = lax.broadcasted_iota(jnp.int32, (E, E), 1)
            ut = jnp.where(eidx < fidx, 1.0, 0.0)
            base_ref[...] = lax.dot_general(
                padded, ut, (((1,), (0,)), ((), ())),
                preferred_element_type=jnp.float32)

        carry_ref[...] = jnp.zeros_like(carry_ref)

    r = lax.broadcasted_iota(jnp.int32, (NPAIR, NPAIR), 0)
    cc = lax.broadcasted_iota(jnp.int32, (NPAIR, NPAIR), 1)
    lt = jnp.where(r > cc, 1.0, 0.0)
    excl = lax.dot_general(lt, oh, (((1,), (0,)), ((), ())),
                           preferred_element_type=jnp.float32)
    excl = excl + carry_ref[...]
    carry_ref[...] = carry_ref[...] + jnp.sum(oh, axis=0, keepdims=True)

    pos = jnp.sum(oh * (excl + base_ref[...]), axis=1, keepdims=True)
    pos_i = pos.astype(jnp.int32)
    pos_ref[...] = jnp.broadcast_to(pos_i, (NPAIR, E))
    w_ref[0] = jnp.concatenate([c1, c2], axis=1)

    @pl.when((p == 1) & (c == NCHUNK - 1))
    def _():
        jj = (lax.broadcasted_iota(jnp.int32, (NT, E), 0) * TG
              ).astype(jnp.float32)
        cnt = jnp.sum(jnp.where(base_ref[...] <= jj, 1, 0),
                      axis=1, keepdims=True) - 1
        te_ref[...] = jnp.broadcast_to(cnt, (NT, E))
        tot = carry_ref[...]
        hit = jnp.where((jj >= base_ref[...]) & (jj < base_ref[...] + tot),
                        1, 0)
        live = jnp.sum(hit, axis=1, keepdims=True)
        live_ref[...] = jnp.broadcast_to(live, (NT, E))


def _route(x, gate_w):
    return pl.pallas_call(
        _route_kernel,
        grid=(2, NCHUNK),
        in_specs=[
            pl.BlockSpec((TM, D), lambda p, c: (c, 0)),
            pl.BlockSpec((E, D), lambda p, c: (0, 0)),
        ],
        out_specs=[
            pl.BlockSpec((NPAIR, E), lambda p, c: (c, 0)),
            pl.BlockSpec((1, TM, TOPK), lambda p, c: (c, 0, 0)),
            pl.BlockSpec((NT, E), lambda p, c: (0, 0)),
            pl.BlockSpec((NT, E), lambda p, c: (0, 0)),
        ],
        out_shape=[
            jax.ShapeDtypeStruct((T * TOPK, E), jnp.int32),
            jax.ShapeDtypeStruct((NCHUNK, TM, TOPK), jnp.float32),
            jax.ShapeDtypeStruct((NT, E), jnp.int32),
            jax.ShapeDtypeStruct((NT, E), jnp.int32),
        ],
        scratch_shapes=[
            pltpu.VMEM((1, E), jnp.float32),
            pltpu.VMEM((1, E), jnp.float32),
        ],
        compiler_params=pltpu.CompilerParams(
            dimension_semantics=("arbitrary", "arbitrary")),
    )(x, gate_w)


PAIRS_PER_W = (T * TOPK) // NW
SCHUNK = 64


def _scatter_x_kernel(pos_hbm, x_hbm, xs_hbm, idx_v, rows_v, sem):
    x_i = x_hbm.bitcast(jnp.int32)
    xs_i = xs_hbm.bitcast(jnp.int32)
    wid = lax.axis_index("s") * NC + lax.axis_index("c")
    for j in range(PAIRS_PER_W // SCHUNK):
        f0 = wid * PAIRS_PER_W + j * SCHUNK
        tok0 = ((f0 >> 9) << 8) + (f0 & 255)
        pltpu.sync_copy(pos_hbm.at[pl.ds(f0, SCHUNK)], idx_v)
        pltpu.sync_copy(x_i.at[pl.ds(tok0, SCHUNK)], rows_v)
        pltpu.async_copy(rows_v, xs_i.at[idx_v], sem).wait()


def _scatter_x(pos_flat, x3):
    return pl.kernel(
        _scatter_x_kernel,
        out_type=jax.ShapeDtypeStruct((P, 16, 128), jnp.bfloat16),
        mesh=plsc.VectorSubcoreMesh(core_axis_name="c", subcore_axis_name="s"),
        scratch_types=[
            pltpu.VMEM((SCHUNK,), jnp.int32),
            pltpu.VMEM((SCHUNK, 8, 128), jnp.int32),
            pltpu.SemaphoreType.DMA,
        ],
    )(pos_flat, x3)


def _gemm_kernel(te_ref, live_ref, xs_ref, wg_ref, wu_ref, wd_ref, ys_ref):
    i = pl.program_id(0)

    @pl.when(live_ref[i] != 0)
    def _():
        xt = xs_ref[...].reshape(TG, D)
        g = lax.dot_general(xt, wg_ref[0], (((1,), (1,)), ((), ())),
                            preferred_element_type=jnp.float32)
        u = lax.dot_general(xt, wu_ref[0], (((1,), (1,)), ((), ())),
                            preferred_element_type=jnp.float32)
        g16 = g.astype(jnp.bfloat16)
        u16 = u.astype(jnp.bfloat16)
        sig = 1.0 / (1.0 + jnp.exp(-g16.astype(jnp.float32)))
        h = ((g16.astype(jnp.float32) * sig).astype(jnp.bfloat16) * u16)
        y = lax.dot_general(h, wd_ref[0], (((1,), (1,)), ((), ())),
                            preferred_element_type=jnp.float32)
        ys_ref[...] = y.astype(jnp.bfloat16).reshape(TG, 16, 128)


def _grouped_gemm(te, live, xs3, w_gate, w_up, w_down):
    grid_spec = pltpu.PrefetchScalarGridSpec(
        num_scalar_prefetch=2,
        grid=(NT,),
        in_specs=[
            pl.BlockSpec((TG, 16, 128), lambda i, te, lv: (i, 0, 0)),
            pl.BlockSpec((1, DFF, D), lambda i, te, lv: (te[i], 0, 0)),
            pl.BlockSpec((1, DFF, D), lambda i, te, lv: (te[i], 0, 0)),
            pl.BlockSpec((1, D, DFF), lambda i, te, lv: (te[i], 0, 0)),
        ],
        out_specs=pl.BlockSpec((TG, 16, 128), lambda i, te, lv: (i, 0, 0)),
    )
    return pl.pallas_call(
        _gemm_kernel,
        grid_spec=grid_spec,
        out_shape=jax.ShapeDtypeStruct((P, 16, 128), jnp.bfloat16),
        compiler_params=pltpu.CompilerParams(
            dimension_semantics=("arbitrary",)),
    )(te, live, xs3, w_gate, w_up, w_down)


TOK_PER_W = T // NW


def _gather_y_kernel(idx0_hbm, idx1_hbm, ys_hbm, r0_hbm, r1_hbm,
                     idx_v, rows_v, sem):
    ys_i = ys_hbm.bitcast(jnp.int32)
    wid = lax.axis_index("s") * NC + lax.axis_index("c")
    base = wid * TOK_PER_W
    for src, dst in ((idx0_hbm, r0_hbm), (idx1_hbm, r1_hbm)):
        pltpu.sync_copy(src.at[pl.ds(base, TOK_PER_W)], idx_v)
        pltpu.async_copy(ys_i.at[idx_v], rows_v, sem).wait()
        pltpu.sync_copy(rows_v, dst.bitcast(jnp.int32).at[pl.ds(base, TOK_PER_W)])


def _gather_y(idx0, idx1, ys3):
    return pl.kernel(
        _gather_y_kernel,
        out_type=[
            jax.ShapeDtypeStruct((T, 16, 128), jnp.bfloat16),
            jax.ShapeDtypeStruct((T, 16, 128), jnp.bfloat16),
        ],
        mesh=plsc.VectorSubcoreMesh(core_axis_name="c", subcore_axis_name="s"),
        scratch_types=[
            pltpu.VMEM((TOK_PER_W,), jnp.int32),
            pltpu.VMEM((TOK_PER_W, 8, 128), jnp.int32),
            pltpu.SemaphoreType.DMA,
        ],
    )(idx0, idx1, ys3)


def _combine_kernel(r0_ref, r1_ref, w_ref, out_ref):
    w0 = w_ref[:, 0:1].astype(jnp.bfloat16)[:, :, None]
    w1 = w_ref[:, 1:2].astype(jnp.bfloat16)[:, :, None]
    out = w0 * r0_ref[...] + w1 * r1_ref[...]
    out_ref[...] = out.reshape(TM, D)


def _combine(r0, r1, wt):
    return pl.pallas_call(
        _combine_kernel,
        grid=(NCHUNK,),
        in_specs=[
            pl.BlockSpec((TM, 16, 128), lambda c: (c, 0, 0)),
            pl.BlockSpec((TM, 16, 128), lambda c: (c, 0, 0)),
            pl.BlockSpec((TM, TOPK), lambda c: (c, 0)),
        ],
        out_specs=pl.BlockSpec((TM, D), lambda c: (c, 0)),
        out_shape=jax.ShapeDtypeStruct((T, D), jnp.bfloat16),
        compiler_params=pltpu.CompilerParams(
            dimension_semantics=("arbitrary",)),
    )(r0, r1, wt)


def kernel(hidden_states, gate_w, w_gate, w_up, w_down):
    b, s, d = hidden_states.shape
    x = hidden_states.reshape(-1, d)

    pos_rep, w_t, te_rep, live_rep = _route(x, gate_w)
    pos_flat = pos_rep[:, 0]
    pos3 = pos_flat.reshape(NCHUNK, TOPK, TM)
    idx0 = pos3[:, 0, :].reshape(T)
    idx1 = pos3[:, 1, :].reshape(T)
    wt = w_t.reshape(T, TOPK)
    te = te_rep[:, 0]
    live = live_rep[:, 0]

    x3 = x.reshape(T, 16, 128)
    xs3 = _scatter_x(pos_flat, x3)
    ys3 = _grouped_gemm(te, live, xs3, w_gate, w_up, w_down)
    r0, r1 = _gather_y(idx0, idx1, ys3)
    out = _combine(r0, r1, wt)
    return out.reshape(b, s, d)


# speedup vs baseline: 5.6115x; 1.0429x over previous
import functools

import jax
import jax.numpy as jnp
from jax import lax
from jax.experimental import pallas as pl
from jax.experimental.pallas import tpu as pltpu
from jax.experimental.pallas import tpu_sc as plsc

E = 8
TOPK = 2
T = 2048
D = 2048
DFF = 768
TM = 256
NCHUNK = T // TM
NPAIR = 2 * TM
TG = 256
P = T * TOPK + E * TG
NT = P // TG

NC = 2
NS = 16
NW = NC * NS


def _top2(x, gate_w):
    logits = lax.dot_general(x, gate_w, (((1,), (1,)), ((), ())),
                             preferred_element_type=jnp.float32)
    logits = logits.astype(jnp.bfloat16).astype(jnp.float32)
    m = jnp.max(logits, axis=1, keepdims=True)
    ex = jnp.exp(logits - m)
    probs = ex / jnp.sum(ex, axis=1, keepdims=True)
    idx = lax.broadcasted_iota(jnp.int32, probs.shape, 1)
    big = jnp.int32(E)
    m1 = jnp.max(probs, axis=1, keepdims=True)
    i1 = jnp.min(jnp.where(probs == m1, idx, big), axis=1, keepdims=True)
    probs2 = jnp.where(idx == i1, -1.0, probs)
    m2 = jnp.max(probs2, axis=1, keepdims=True)
    i2 = jnp.min(jnp.where(probs2 == m2, idx, big), axis=1, keepdims=True)
    s = m1 + m2
    c1 = (m1 / s).astype(jnp.bfloat16).astype(jnp.float32)
    c2 = (m2 / s).astype(jnp.bfloat16).astype(jnp.float32)
    oh1 = jnp.where(idx == i1, 1.0, 0.0)
    oh2 = jnp.where(idx == i2, 1.0, 0.0)
    return c1, c2, oh1, oh2


def _route_kernel(x_ref, gate_ref, pos_ref, w_ref, te_ref, live_ref,
                  carry_ref, base_ref, oh_ref, cw_ref):
    p = pl.program_id(0)
    c = pl.program_id(1)

    @pl.when(p == 0)
    def _():
        c1, c2, oh1, oh2 = _top2(x_ref[...], gate_ref[...])
        oh_ref[c] = jnp.concatenate([oh1, oh2], axis=0)
        cw_ref[c] = jnp.concatenate([c1, c2], axis=1)

    oh = oh_ref[c]

    @pl.when(c == 0)
    def _():
        @pl.when(p == 0)
        def _():
            base_ref[...] = jnp.zeros_like(base_ref)

        @pl.when(p == 1)
        def _():
            totals = carry_ref[...]
            padded = jnp.ceil(totals / TG) * TG
            eidx = lax.broadcasted_iota(jnp.int32, (E, E), 0)
            fidx = lax.broadcasted_iota(jnp.int32, (E, E), 1)
            ut = jnp.where(eidx < fidx, 1.0, 0.0)
            base_ref[...] = lax.dot_general(
                padded, ut, (((1,), (0,)), ((), ())),
                preferred_element_type=jnp.float32)

        carry_ref[...] = jnp.zeros_like(carry_ref)

    r = lax.broadcasted_iota(jnp.int32, (NPAIR, NPAIR), 0)
    cc = lax.broadcasted_iota(jnp.int32, (NPAIR, NPAIR), 1)
    lt = jnp.where(r > cc, 1.0, 0.0)
    excl = lax.dot_general(lt, oh, (((1,), (0,)), ((), ())),
                           preferred_element_type=jnp.float32)
    excl = excl + carry_ref[...]
    carry_ref[...] = carry_ref[...] + jnp.sum(oh, axis=0, keepdims=True)

    pos = jnp.sum(oh * (excl + base_ref[...]), axis=1, keepdims=True)
    pos_i = pos.astype(jnp.int32)
    pos_ref[...] = jnp.broadcast_to(pos_i, (NPAIR, E))
    w_ref[0] = cw_ref[c]

    @pl.when((p == 1) & (c == NCHUNK - 1))
    def _():
        jj = (lax.broadcasted_iota(jnp.int32, (NT, E), 0) * TG
              ).astype(jnp.float32)
        cnt = jnp.sum(jnp.where(base_ref[...] <= jj, 1, 0),
                      axis=1, keepdims=True) - 1
        te_ref[...] = jnp.broadcast_to(cnt, (NT, E))
        tot = carry_ref[...]
        hit = jnp.where((jj >= base_ref[...]) & (jj < base_ref[...] + tot),
                        1, 0)
        live = jnp.sum(hit, axis=1, keepdims=True)
        live_ref[...] = jnp.broadcast_to(live, (NT, E))


def _route(x, gate_w):
    return pl.pallas_call(
        _route_kernel,
        grid=(2, NCHUNK),
        in_specs=[
            pl.BlockSpec((TM, D), lambda p, c: (c * (1 - p), 0)),
            pl.BlockSpec((E, D), lambda p, c: (0, 0)),
        ],
        out_specs=[
            pl.BlockSpec((NPAIR, E), lambda p, c: (c, 0)),
            pl.BlockSpec((1, TM, TOPK), lambda p, c: (c, 0, 0)),
            pl.BlockSpec((NT, E), lambda p, c: (0, 0)),
            pl.BlockSpec((NT, E), lambda p, c: (0, 0)),
        ],
        out_shape=[
            jax.ShapeDtypeStruct((T * TOPK, E), jnp.int32),
            jax.ShapeDtypeStruct((NCHUNK, TM, TOPK), jnp.float32),
            jax.ShapeDtypeStruct((NT, E), jnp.int32),
            jax.ShapeDtypeStruct((NT, E), jnp.int32),
        ],
        scratch_shapes=[
            pltpu.VMEM((1, E), jnp.float32),
            pltpu.VMEM((1, E), jnp.float32),
            pltpu.VMEM((NCHUNK, NPAIR, E), jnp.float32),
            pltpu.VMEM((NCHUNK, TM, TOPK), jnp.float32),
        ],
        compiler_params=pltpu.CompilerParams(
            dimension_semantics=("arbitrary", "arbitrary")),
    )(x, gate_w)


PAIRS_PER_W = (T * TOPK) // NW
SCHUNK = 64


def _scatter_x_kernel(pos_hbm, x_hbm, xs_hbm, idx_v, rows_v, sem):
    x_i = x_hbm.bitcast(jnp.int32)
    xs_i = xs_hbm.bitcast(jnp.int32)
    wid = lax.axis_index("s") * NC + lax.axis_index("c")
    for j in range(PAIRS_PER_W // SCHUNK):
        f0 = wid * PAIRS_PER_W + j * SCHUNK
        tok0 = ((f0 >> 9) << 8) + (f0 & 255)
        pltpu.sync_copy(pos_hbm.at[pl.ds(f0, SCHUNK)], idx_v)
        pltpu.sync_copy(x_i.at[pl.ds(tok0, SCHUNK)], rows_v)
        pltpu.async_copy(rows_v, xs_i.at[idx_v], sem).wait()


def _scatter_x(pos_flat, x3):
    return pl.kernel(
        _scatter_x_kernel,
        out_type=jax.ShapeDtypeStruct((P, 16, 128), jnp.bfloat16),
        mesh=plsc.VectorSubcoreMesh(core_axis_name="c", subcore_axis_name="s"),
        scratch_types=[
            pltpu.VMEM((SCHUNK,), jnp.int32),
            pltpu.VMEM((SCHUNK, 8, 128), jnp.int32),
            pltpu.SemaphoreType.DMA,
        ],
    )(pos_flat, x3)


def _gemm_kernel(te_ref, live_ref, xs_ref, wg_ref, wu_ref, wd_ref, ys_ref):
    i = pl.program_id(0)

    @pl.when(live_ref[i] != 0)
    def _():
        xt = xs_ref[...].reshape(TG, D)
        g = lax.dot_general(xt, wg_ref[0], (((1,), (1,)), ((), ())),
                            preferred_element_type=jnp.float32)
        u = lax.dot_general(xt, wu_ref[0], (((1,), (1,)), ((), ())),
                            preferred_element_type=jnp.float32)
        g16 = g.astype(jnp.bfloat16)
        u16 = u.astype(jnp.bfloat16)
        sig = 1.0 / (1.0 + jnp.exp(-g16.astype(jnp.float32)))
        h = ((g16.astype(jnp.float32) * sig).astype(jnp.bfloat16) * u16)
        y = lax.dot_general(h, wd_ref[0], (((1,), (1,)), ((), ())),
                            preferred_element_type=jnp.float32)
        ys_ref[...] = y.astype(jnp.bfloat16).reshape(TG, 16, 128)


def _grouped_gemm(te, live, xs3, w_gate, w_up, w_down):
    grid_spec = pltpu.PrefetchScalarGridSpec(
        num_scalar_prefetch=2,
        grid=(NT,),
        in_specs=[
            pl.BlockSpec((TG, 16, 128), lambda i, te, lv: (i, 0, 0)),
            pl.BlockSpec((1, DFF, D), lambda i, te, lv: (te[i], 0, 0)),
            pl.BlockSpec((1, DFF, D), lambda i, te, lv: (te[i], 0, 0)),
            pl.BlockSpec((1, D, DFF), lambda i, te, lv: (te[i], 0, 0)),
        ],
        out_specs=pl.BlockSpec((TG, 16, 128), lambda i, te, lv: (i, 0, 0)),
    )
    return pl.pallas_call(
        _gemm_kernel,
        grid_spec=grid_spec,
        out_shape=jax.ShapeDtypeStruct((P, 16, 128), jnp.bfloat16),
        compiler_params=pltpu.CompilerParams(
            dimension_semantics=("arbitrary",)),
    )(te, live, xs3, w_gate, w_up, w_down)


TOK_PER_W = T // NW


def _gather_y_kernel(idx0_hbm, idx1_hbm, ys_hbm, r0_hbm, r1_hbm,
                     idx_v, rows_v, sem):
    ys_i = ys_hbm.bitcast(jnp.int32)
    wid = lax.axis_index("s") * NC + lax.axis_index("c")
    base = wid * TOK_PER_W
    for src, dst in ((idx0_hbm, r0_hbm), (idx1_hbm, r1_hbm)):
        pltpu.sync_copy(src.at[pl.ds(base, TOK_PER_W)], idx_v)
        pltpu.async_copy(ys_i.at[idx_v], rows_v, sem).wait()
        pltpu.sync_copy(rows_v, dst.bitcast(jnp.int32).at[pl.ds(base, TOK_PER_W)])


def _gather_y(idx0, idx1, ys3):
    return pl.kernel(
        _gather_y_kernel,
        out_type=[
            jax.ShapeDtypeStruct((T, 16, 128), jnp.bfloat16),
            jax.ShapeDtypeStruct((T, 16, 128), jnp.bfloat16),
        ],
        mesh=plsc.VectorSubcoreMesh(core_axis_name="c", subcore_axis_name="s"),
        scratch_types=[
            pltpu.VMEM((TOK_PER_W,), jnp.int32),
            pltpu.VMEM((TOK_PER_W, 8, 128), jnp.int32),
            pltpu.SemaphoreType.DMA,
        ],
    )(idx0, idx1, ys3)


def _combine_kernel(r0_ref, r1_ref, w_ref, out_ref):
    w0 = w_ref[:, 0:1].astype(jnp.bfloat16)[:, :, None]
    w1 = w_ref[:, 1:2].astype(jnp.bfloat16)[:, :, None]
    out = w0 * r0_ref[...] + w1 * r1_ref[...]
    out_ref[...] = out.reshape(TM, D)


def _combine(r0, r1, wt):
    return pl.pallas_call(
        _combine_kernel,
        grid=(NCHUNK,),
        in_specs=[
            pl.BlockSpec((TM, 16, 128), lambda c: (c, 0, 0)),
            pl.BlockSpec((TM, 16, 128), lambda c: (c, 0, 0)),
            pl.BlockSpec((TM, TOPK), lambda c: (c, 0)),
        ],
        out_specs=pl.BlockSpec((TM, D), lambda c: (c, 0)),
        out_shape=jax.ShapeDtypeStruct((T, D), jnp.bfloat16),
        compiler_params=pltpu.CompilerParams(
            dimension_semantics=("arbitrary",)),
    )(r0, r1, wt)


def kernel(hidden_states, gate_w, w_gate, w_up, w_down):
    b, s, d = hidden_states.shape
    x = hidden_states.reshape(-1, d)

    pos_rep, w_t, te_rep, live_rep = _route(x, gate_w)
    pos_flat = pos_rep[:, 0]
    pos3 = pos_flat.reshape(NCHUNK, TOPK, TM)
    idx0 = pos3[:, 0, :].reshape(T)
    idx1 = pos3[:, 1, :].reshape(T)
    wt = w_t.reshape(T, TOPK)
    te = te_rep[:, 0]
    live = live_rep[:, 0]

    x3 = x.reshape(T, 16, 128)
    xs3 = _scatter_x(pos_flat, x3)
    ys3 = _grouped_gemm(te, live, xs3, w_gate, w_up, w_down)
    r0, r1 = _gather_y(idx0, idx1, ys3)
    out = _combine(r0, r1, wt)
    return out.reshape(b, s, d)
